# baseline (speedup 1.0000x reference)
import functools

import jax
import jax.numpy as jnp
from jax import lax
from jax.experimental import pallas as pl
from jax.experimental.pallas import tpu as pltpu
from jax.experimental.pallas import tpu_sc as plsc


@functools.cache
def _make_sc_gather(V, D, B):
    info = plsc.get_sparse_core_info()
    NC, NS = info.num_cores, info.num_subcores
    NW = NC * NS
    assert B % (NW * 16) == 0
    b_per_w = B // NW
    n_grp = b_per_w // 16
    mesh = plsc.VectorSubcoreMesh(core_axis_name="c", subcore_axis_name="s")

    @functools.partial(
        pl.kernel,
        mesh=mesh,
        out_type=jax.ShapeDtypeStruct((B, D), jnp.float32),
        compiler_params=pltpu.CompilerParams(
            use_tc_tiling_on_sc=True, disable_bounds_checks=True,
            skip_device_barrier=True),
        scratch_types=[
            pltpu.VMEM((b_per_w,), jnp.int32),
            pltpu.VMEM((b_per_w, D), jnp.float32),
            pltpu.SemaphoreType.DMA,
        ],
    )
    def sc_gather(table_hbm, idx_hbm, out_hbm, idx_v, rows_v, sem):
        wid = lax.axis_index("s") * NC + lax.axis_index("c")
        base = wid * b_per_w
        pltpu.sync_copy(idx_hbm.at[pl.ds(base, b_per_w)], idx_v)

        def group(g, carry):
            vec = idx_v[pl.ds(g * 16, 16)]
            for l in range(16):
                row = vec[l]
                pltpu.make_async_copy(
                    table_hbm.at[pl.ds(row, 1)],
                    rows_v.at[pl.ds(g * 16 + l, 1)],
                    sem,
                ).start()
            return carry

        lax.fori_loop(0, n_grp, group, 0)
        pltpu.make_async_copy(
            table_hbm.at[pl.ds(0, b_per_w)], rows_v, sem).wait()
        pltpu.sync_copy(rows_v, out_hbm.at[pl.ds(base, b_per_w)])

    return sc_gather


@functools.cache
def _make_tc_project(B, K, N, BM):
    def body(w_ref, x_ref, b_ref, o_ref):
        acc = jax.lax.dot_general(
            w_ref[...], x_ref[...], (((1,), (1,)), ((), ())),
            preferred_element_type=jnp.float32)
        o_ref[...] = jnp.maximum(acc + b_ref[...], 0.0)

    return pl.pallas_call(
        body,
        grid=(B // BM,),
        in_specs=[
            pl.BlockSpec((N, K), lambda i: (0, 0)),
            pl.BlockSpec((BM, K), lambda i: (i, 0)),
            pl.BlockSpec((N, 1), lambda i: (0, 0)),
        ],
        out_specs=pl.BlockSpec((N, BM), lambda i: (0, i)),
        out_shape=jax.ShapeDtypeStruct((N, B), jnp.float32),
    )


@functools.cache
def _make_tc_project_half(B, K, N, half, aliased):
    BM = B // 2

    def body(*refs):
        w_ref, x_ref, b_ref, o_ref = refs[-4:]
        acc = jax.lax.dot_general(
            w_ref[...], x_ref[...], (((1,), (1,)), ((), ())),
            preferred_element_type=jnp.float32)
        o_ref[...] = jnp.maximum(acc + b_ref[...], 0.0)

    in_specs = [
        pl.BlockSpec((N, K), lambda i: (0, 0)),
        pl.BlockSpec((BM, K), lambda i: (0, 0)),
        pl.BlockSpec((N, 1), lambda i: (0, 0)),
    ]
    if aliased:
        in_specs = [pl.BlockSpec(memory_space=pltpu.MemorySpace.HBM)] + in_specs

    return pl.pallas_call(
        body,
        grid=(1,),
        in_specs=in_specs,
        out_specs=pl.BlockSpec((N, BM), lambda i: (0, half)),
        out_shape=jax.ShapeDtypeStruct((N, B), jnp.float32),
        input_output_aliases={0: 0} if aliased else {},
    )


def kernel(inputs, table, W, b):
    B = inputs.shape[0]
    V, D = table.shape
    N = W.shape[1]
    H = B // 2
    idx = inputs.reshape(B).astype(jnp.int32)
    W_t = W.T
    b_c = b.reshape(N, 1)
    ga = _make_sc_gather(V, D, H)(table, idx[:H])
    gb = _make_sc_gather(V, D, H)(table, idx[H:])
    out_t = _make_tc_project_half(B, D, N, 0, False)(W_t, ga, b_c)
    out_t = _make_tc_project_half(B, D, N, 1, True)(out_t, W_t, gb, b_c)
    return out_t.T


# speedup vs baseline: 1.0715x; 1.0715x over previous
import functools

import jax
import jax.numpy as jnp
from jax import lax
from jax.experimental import pallas as pl
from jax.experimental.pallas import tpu as pltpu
from jax.experimental.pallas import tpu_sc as plsc


@functools.cache
def _make_sc_gather(V, D, B):
    info = plsc.get_sparse_core_info()
    NC, NS = info.num_cores, info.num_subcores
    NW = NC * NS
    assert B % (NW * 16) == 0
    b_per_w = B // NW
    n_grp = b_per_w // 16
    mesh = plsc.VectorSubcoreMesh(core_axis_name="c", subcore_axis_name="s")

    @functools.partial(
        pl.kernel,
        mesh=mesh,
        out_type=jax.ShapeDtypeStruct((B, D), jnp.float32),
        compiler_params=pltpu.CompilerParams(
            use_tc_tiling_on_sc=True, disable_bounds_checks=True,
            skip_device_barrier=True),
        scratch_types=[
            pltpu.VMEM((b_per_w,), jnp.int32),
            pltpu.VMEM((b_per_w, D), jnp.float32),
            pltpu.SemaphoreType.DMA,
        ],
    )
    def sc_gather(table_hbm, idx_hbm, out_hbm, idx_v, rows_v, sem):
        wid = lax.axis_index("s") * NC + lax.axis_index("c")
        base = wid * b_per_w
        pltpu.sync_copy(idx_hbm.at[pl.ds(base, b_per_w)], idx_v)

        def group(g, carry):
            vec = idx_v[pl.ds(g * 16, 16)]
            for l in range(16):
                row = vec[l]
                pltpu.make_async_copy(
                    table_hbm.at[pl.ds(row, 1)],
                    rows_v.at[pl.ds(g * 16 + l, 1)],
                    sem,
                ).start()
            return carry

        lax.fori_loop(0, n_grp, group, 0)
        pltpu.make_async_copy(
            table_hbm.at[pl.ds(0, b_per_w)], rows_v, sem).wait()
        pltpu.sync_copy(rows_v, out_hbm.at[pl.ds(base, b_per_w)])

    return sc_gather


@functools.cache
def _make_tc_project(B, K, N, BM):
    def body(w_ref, x_ref, b_ref, o_ref):
        acc = jax.lax.dot_general(
            w_ref[...], x_ref[...], (((1,), (1,)), ((), ())),
            preferred_element_type=jnp.float32)
        o_ref[...] = jnp.maximum(acc + b_ref[...], 0.0)

    return pl.pallas_call(
        body,
        grid=(B // BM,),
        in_specs=[
            pl.BlockSpec((N, K), lambda i: (0, 0)),
            pl.BlockSpec((BM, K), lambda i: (i, 0)),
            pl.BlockSpec((N, 1), lambda i: (0, 0)),
        ],
        out_specs=pl.BlockSpec((N, BM), lambda i: (0, i)),
        out_shape=jax.ShapeDtypeStruct((N, B), jnp.float32),
    )


def kernel(inputs, table, W, b):
    B = inputs.shape[0]
    V, D = table.shape
    N = W.shape[1]
    idx = inputs.reshape(B).astype(jnp.int32)
    gathered = _make_sc_gather(V, D, B)(table, idx)
    out_t = _make_tc_project(B, D, N, 8192)(W.T, gathered, b.reshape(N, 1))
    return out_t.T


# speedup vs baseline: 1.2308x; 1.1486x over previous
import functools

import jax
import jax.numpy as jnp
from jax import lax
from jax.experimental import pallas as pl
from jax.experimental.pallas import tpu as pltpu
from jax.experimental.pallas import tpu_sc as plsc


@functools.cache
def _make_sc_gather(V, D, B):
    info = plsc.get_sparse_core_info()
    NC, NS = info.num_cores, info.num_subcores
    NW = NC * NS
    assert B % (NW * 16) == 0
    b_per_w = B // NW
    n_grp = b_per_w // 16
    mesh = plsc.VectorSubcoreMesh(core_axis_name="c", subcore_axis_name="s")

    @functools.partial(
        pl.kernel,
        mesh=mesh,
        out_type=jax.ShapeDtypeStruct((B, D), jnp.float32),
        compiler_params=pltpu.CompilerParams(
            use_tc_tiling_on_sc=True, disable_bounds_checks=True,
            skip_device_barrier=True),
        scratch_types=[
            pltpu.VMEM((b_per_w,), jnp.int32),
            pltpu.VMEM((b_per_w, D), jnp.float32),
            pltpu.SemaphoreType.DMA,
        ],
    )
    def sc_gather(table_hbm, idx_hbm, out_hbm, idx_v, rows_v, sem):
        wid = lax.axis_index("s") * NC + lax.axis_index("c")
        base = wid * b_per_w
        pltpu.sync_copy(idx_hbm.at[pl.ds(base, b_per_w)], idx_v)

        def group(g, carry):
            vec = idx_v[pl.ds(g * 16, 16)]
            for l in range(16):
                row = vec[l]
                pltpu.make_async_copy(
                    table_hbm.at[pl.ds(row, 1)],
                    rows_v.at[pl.ds(g * 16 + l, 1)],
                    sem,
                ).start()
            return carry

        lax.fori_loop(0, n_grp, group, 0)
        pltpu.make_async_copy(
            table_hbm.at[pl.ds(0, b_per_w)], rows_v, sem).wait()
        pltpu.sync_copy(rows_v, out_hbm.at[pl.ds(base, b_per_w)])

    return sc_gather


@functools.cache
def _make_tc_transpose(D, V, BV):
    grid = (V + BV - 1) // BV

    def body(x_ref, o_ref):
        o_ref[...] = x_ref[...].T

    return pl.pallas_call(
        body,
        grid=(grid,),
        in_specs=[pl.BlockSpec((D, BV), lambda i: (0, i))],
        out_specs=pl.BlockSpec((BV, D), lambda i: (i, 0)),
        out_shape=jax.ShapeDtypeStruct((V, D), jnp.float32),
    )


@functools.cache
def _make_tc_project(B, K, N, BM):
    def body(w_ref, x_ref, b_ref, o_ref):
        acc = jax.lax.dot_general(
            w_ref[...], x_ref[...], (((1,), (1,)), ((), ())),
            preferred_element_type=jnp.float32)
        o_ref[...] = jnp.maximum(acc + b_ref[...], 0.0)

    return pl.pallas_call(
        body,
        grid=(B // BM,),
        in_specs=[
            pl.BlockSpec((N, K), lambda i: (0, 0)),
            pl.BlockSpec((BM, K), lambda i: (i, 0)),
            pl.BlockSpec((N, 1), lambda i: (0, 0)),
        ],
        out_specs=pl.BlockSpec((N, BM), lambda i: (0, i)),
        out_shape=jax.ShapeDtypeStruct((N, B), jnp.float32),
    )


def kernel(inputs, table, W, b):
    B = inputs.shape[0]
    V, D = table.shape
    N = W.shape[1]
    idx = inputs.reshape(B).astype(jnp.int32)
    table_rm = _make_tc_transpose(D, V, 8192)(table.T)
    gathered = _make_sc_gather(V, D, B)(table_rm, idx)
    out_t = _make_tc_project(B, D, N, 8192)(W.T, gathered, b.reshape(N, 1))
    return out_t.T


# speedup vs baseline: 1.2492x; 1.0150x over previous
import functools

import jax
import jax.numpy as jnp
from jax import lax
from jax.experimental import pallas as pl
from jax.experimental.pallas import tpu as pltpu
from jax.experimental.pallas import tpu_sc as plsc


@functools.cache
def _make_sc_gather(V, D, B):
    info = plsc.get_sparse_core_info()
    NC, NS = info.num_cores, info.num_subcores
    NW = NC * NS
    assert B % (NW * 16) == 0
    b_per_w = B // NW
    n_grp = b_per_w // 16
    mesh = plsc.VectorSubcoreMesh(core_axis_name="c", subcore_axis_name="s")

    @functools.partial(
        pl.kernel,
        mesh=mesh,
        out_type=jax.ShapeDtypeStruct((B, D), jnp.float32),
        compiler_params=pltpu.CompilerParams(
            use_tc_tiling_on_sc=True, disable_bounds_checks=True,
            skip_device_barrier=True),
        scratch_types=[
            pltpu.VMEM((b_per_w,), jnp.int32),
            pltpu.VMEM((b_per_w, D), jnp.float32),
            pltpu.SemaphoreType.DMA,
        ],
    )
    def sc_gather(table_hbm, idx_hbm, out_hbm, idx_v, rows_v, sem):
        wid = lax.axis_index("s") * NC + lax.axis_index("c")
        base = wid * b_per_w
        pltpu.sync_copy(idx_hbm.at[pl.ds(base, b_per_w)], idx_v)

        def group(g, carry):
            vec = idx_v[pl.ds(g * 16, 16)]
            for l in range(16):
                row = vec[l]
                pltpu.make_async_copy(
                    table_hbm.at[pl.ds(row, 1)],
                    rows_v.at[pl.ds(g * 16 + l, 1)],
                    sem,
                ).start()
            return carry

        lax.fori_loop(0, n_grp, group, 0)
        pltpu.make_async_copy(
            table_hbm.at[pl.ds(0, b_per_w)], rows_v, sem).wait()
        pltpu.sync_copy(rows_v, out_hbm.at[pl.ds(base, b_per_w)])

    return sc_gather


@functools.cache
def _make_tc_transpose(D, V, BV):
    grid = (V + BV - 1) // BV

    def body(x_ref, o_ref):
        o_ref[...] = x_ref[...].T

    return pl.pallas_call(
        body,
        grid=(grid,),
        in_specs=[pl.BlockSpec((D, BV), lambda i: (0, i))],
        out_specs=pl.BlockSpec((BV, D), lambda i: (i, 0)),
        out_shape=jax.ShapeDtypeStruct((V, D), jnp.float32),
    )


@functools.cache
def _make_tc_project(B, K, N, BM):
    def body(w_ref, x_ref, b_ref, o_ref):
        acc = jax.lax.dot_general(
            w_ref[...], x_ref[...], (((1,), (1,)), ((), ())),
            preferred_element_type=jnp.float32)
        o_ref[...] = jnp.maximum(acc + b_ref[...], 0.0)

    return pl.pallas_call(
        body,
        grid=(B // BM,),
        in_specs=[
            pl.BlockSpec((N, K), lambda i: (0, 0)),
            pl.BlockSpec((BM, K), lambda i: (i, 0)),
            pl.BlockSpec((N, 1), lambda i: (0, 0)),
        ],
        out_specs=pl.BlockSpec((N, BM), lambda i: (0, i)),
        out_shape=jax.ShapeDtypeStruct((N, B), jnp.float32),
    )


def kernel(inputs, table, W, b):
    B = inputs.shape[0]
    V, D = table.shape
    N = W.shape[1]
    idx = inputs.reshape(B).astype(jnp.int32)
    table_rm = _make_tc_transpose(D, V, 16384)(table.T)
    gathered = _make_sc_gather(V, D, B)(table_rm, idx)
    out_t = _make_tc_project(B, D, N, 8192)(W.T, gathered, b.reshape(N, 1))
    return out_t.T


# speedup vs baseline: 1.2536x; 1.0035x over previous
import functools

import jax
import jax.numpy as jnp
from jax import lax
from jax.experimental import pallas as pl
from jax.experimental.pallas import tpu as pltpu
from jax.experimental.pallas import tpu_sc as plsc


@functools.cache
def _make_sc_gather(V, D, B):
    info = plsc.get_sparse_core_info()
    NC, NS = info.num_cores, info.num_subcores
    NW = NC * NS
    assert B % (NW * 16) == 0
    b_per_w = B // NW
    n_grp = b_per_w // 16
    mesh = plsc.VectorSubcoreMesh(core_axis_name="c", subcore_axis_name="s")

    @functools.partial(
        pl.kernel,
        mesh=mesh,
        out_type=jax.ShapeDtypeStruct((B, D), jnp.float32),
        compiler_params=pltpu.CompilerParams(
            use_tc_tiling_on_sc=True, disable_bounds_checks=True,
            skip_device_barrier=True),
        scratch_types=[
            pltpu.VMEM((b_per_w,), jnp.int32),
            pltpu.VMEM((b_per_w, D), jnp.float32),
            pltpu.SemaphoreType.DMA,
        ],
    )
    def sc_gather(table_hbm, idx_hbm, out_hbm, idx_v, rows_v, sem):
        wid = lax.axis_index("s") * NC + lax.axis_index("c")
        base = wid * b_per_w
        pltpu.sync_copy(idx_hbm.at[pl.ds(base, b_per_w)], idx_v)

        def group(g, carry):
            vec = idx_v[pl.ds(g * 16, 16)]
            for l in range(16):
                row = vec[l]
                pltpu.make_async_copy(
                    table_hbm.at[pl.ds(row, 1)],
                    rows_v.at[pl.ds(g * 16 + l, 1)],
                    sem,
                ).start()
            return carry

        lax.fori_loop(0, n_grp, group, 0)
        pltpu.make_async_copy(
            table_hbm.at[pl.ds(0, b_per_w)], rows_v, sem).wait()
        pltpu.sync_copy(rows_v, out_hbm.at[pl.ds(base, b_per_w)])

    return sc_gather


@functools.cache
def _make_tc_transpose(D, V, BV):
    grid = (V + BV - 1) // BV

    def body(x_ref, o_ref):
        o_ref[...] = x_ref[...].T

    return pl.pallas_call(
        body,
        grid=(grid,),
        in_specs=[pl.BlockSpec((D, BV), lambda i: (0, i))],
        out_specs=pl.BlockSpec((BV, D), lambda i: (i, 0)),
        out_shape=jax.ShapeDtypeStruct((V, D), jnp.float32),
    )


@functools.cache
def _make_tc_project(B, K, N, BM):
    def body(w_ref, x_ref, b_ref, o_ref):
        acc = jax.lax.dot_general(
            w_ref[...], x_ref[...], (((1,), (1,)), ((), ())),
            preferred_element_type=jnp.float32)
        o_ref[...] = jnp.maximum(acc + b_ref[...], 0.0)

    return pl.pallas_call(
        body,
        grid=(B // BM,),
        in_specs=[
            pl.BlockSpec((N, K), lambda i: (0, 0)),
            pl.BlockSpec((BM, K), lambda i: (i, 0)),
            pl.BlockSpec((N, 1), lambda i: (0, 0)),
        ],
        out_specs=pl.BlockSpec((N, BM), lambda i: (0, i)),
        out_shape=jax.ShapeDtypeStruct((N, B), jnp.float32),
    )


def kernel(inputs, table, W, b):
    B = inputs.shape[0]
    V, D = table.shape
    N = W.shape[1]
    idx = inputs.reshape(B).astype(jnp.int32)
    table_rm = _make_tc_transpose(D, V, 25088)(table.T)
    gathered = _make_sc_gather(V, D, B)(table_rm, idx)
    out_t = _make_tc_project(B, D, N, 8192)(W.T, gathered, b.reshape(N, 1))
    return out_t.T
